# baseline (speedup 1.0000x reference)
import functools

import jax
import jax.numpy as jnp
from jax import lax
from jax.experimental import pallas as pl
from jax.experimental.pallas import tpu as pltpu
from jax.experimental.pallas import tpu_sc as plsc

B = 16384
NUM_NUMERIC = 13
NUM_CAT = 26
VOCAB = 100000
EMB = 16
H1 = 256
H2 = 128
EPS = 1e-5

TOT = B * NUM_CAT
NC, NS = 2, 16
NW = NC * NS
LPW = TOT // NW
GPW = LPW // 128
SPW = B // NW
FLUSH = 13
ROWS_F = FLUSH * 128 // NUM_CAT
XD = 512
TAB_R = NUM_CAT * VOCAB // 8


@functools.cache
def _make_sc_gather():
    mesh = plsc.VectorSubcoreMesh(core_axis_name="c", subcore_axis_name="s")

    @functools.partial(
        pl.kernel,
        out_type=jax.ShapeDtypeStruct((B, XD), jnp.float32),
        mesh=mesh,
        scratch_types=[
            pltpu.VMEM((GPW, 128), jnp.int32),
            pltpu.VMEM((GPW, 128), jnp.int32),
            pltpu.VMEM((2, 128, 128), jnp.float32),
            pltpu.VMEM((ROWS_F, XD), jnp.float32),
            pltpu.SemaphoreType.DMA((2,)),
        ],
        compiler_params=pltpu.CompilerParams(needs_layout_passes=False),
    )
    def _sc_gather(tab_hbm, idx_hbm, out_hbm, idx_v, idxr_v, rows_v, asm_v,
                   sems):
        wid = lax.axis_index("s") * NC + lax.axis_index("c")
        pltpu.sync_copy(idx_hbm.at[wid], idx_v)

        iota = jnp.arange(16, dtype=jnp.int32)
        zeros = jnp.zeros((16,), jnp.float32)

        def zero_body(r, carry):
            for c in range(EMB * NUM_CAT // 16, XD // 16):
                asm_v[r, pl.ds(c * 16, 16)] = zeros
            return carry

        lax.fori_loop(0, ROWS_F, zero_body, 0)

        def shift_body(g, carry):
            for c in range(8):
                idxr_v[g, pl.ds(c * 16, 16)] = (
                    idx_v[g, pl.ds(c * 16, 16)] >> 3)
            return carry

        lax.fori_loop(0, GPW, shift_body, 0)

        def fire(g, slot):
            return pltpu.async_copy(
                tab_hbm.at[idxr_v.at[g]], rows_v.at[slot], sems.at[slot])

        fire(0, 0)

        def group_body(g, carry):
            slot = lax.rem(g, 2)

            @pl.when(g < GPW - 1)
            def _():
                fire(g + 1, lax.rem(g + 1, 2))

            pltpu.make_async_copy(
                tab_hbm.at[idxr_v.at[g]], rows_v.at[slot],
                sems.at[slot]).wait()

            base = lax.rem(g, FLUSH) * 128
            arow_base = jnp.full((16,), 0, jnp.int32)
            slotv = jnp.full((16,), slot, jnp.int32)
            for c in range(8):
                flat = base + c * 16 + iota
                idxc = idx_v[g, pl.ds(c * 16, 16)]
                off = (idxc & 7) << 4
                srow = flat // NUM_CAT
                scol = lax.rem(flat, NUM_CAT) << 4
                jvec = jnp.full((16,), c * 16, jnp.int32) + iota
                for e in range(EMB):
                    vals = plsc.load_gather(rows_v, [slotv, jvec, off + e])
                    plsc.store_scatter(asm_v, [srow, scol + e], vals)

            @pl.when(lax.rem(g, FLUSH) == FLUSH - 1)
            def _():
                s = g // FLUSH
                pltpu.sync_copy(
                    asm_v,
                    out_hbm.at[pl.ds(wid * SPW + s * ROWS_F, ROWS_F)])

            return carry

        lax.fori_loop(0, GPW, group_body, 0)

    return _sc_gather


BS = 1024
NB = B // BS
_INV_B = 1.0 / B
_F32 = jnp.float32


def _a_body(nx_ref, em_ref, w1a_ref, w1b_ref, b1_ref, wwa_ref, wwb_ref,
            h1_ref, wide_ref, s1_ref, s2_ref):
    i = pl.program_id(0)
    nx = nx_ref[...]
    em = em_ref[...]
    h = (jnp.dot(nx, w1a_ref[...], preferred_element_type=_F32)
         + jnp.dot(em, w1b_ref[...], preferred_element_type=_F32)
         + b1_ref[...])
    h1_ref[...] = h
    wide_ref[...] = (jnp.sum(nx * wwa_ref[...], axis=1)
                     + jnp.sum(em * wwb_ref[...], axis=1))
    s1 = jnp.sum(h, axis=0, keepdims=True)
    s2 = jnp.sum(h * h, axis=0, keepdims=True)

    @pl.when(i == 0)
    def _():
        s1_ref[...] = s1
        s2_ref[...] = s2

    @pl.when(i > 0)
    def _():
        s1_ref[...] += s1
        s2_ref[...] += s2


_a_call = pl.pallas_call(
    _a_body,
    grid=(NB,),
    in_specs=[
        pl.BlockSpec((BS, NUM_NUMERIC), lambda i: (i, 0)),
        pl.BlockSpec((BS, XD), lambda i: (i, 0)),
        pl.BlockSpec((NUM_NUMERIC, H1), lambda i: (0, 0)),
        pl.BlockSpec((XD, H1), lambda i: (0, 0)),
        pl.BlockSpec((H1,), lambda i: (0,)),
        pl.BlockSpec((1, NUM_NUMERIC), lambda i: (0, 0)),
        pl.BlockSpec((1, XD), lambda i: (0, 0)),
    ],
    out_specs=[
        pl.BlockSpec((BS, H1), lambda i: (i, 0)),
        pl.BlockSpec((BS,), lambda i: (i,)),
        pl.BlockSpec((1, H1), lambda i: (0, 0)),
        pl.BlockSpec((1, H1), lambda i: (0, 0)),
    ],
    out_shape=[
        jax.ShapeDtypeStruct((B, H1), _F32),
        jax.ShapeDtypeStruct((B,), _F32),
        jax.ShapeDtypeStruct((1, H1), _F32),
        jax.ShapeDtypeStruct((1, H1), _F32),
    ],
)


def _b_body(h1_ref, s1_ref, s2_ref, g1_ref, be1_ref, w2_ref, b2_ref,
            h2_ref, t1_ref, t2_ref):
    i = pl.program_id(0)
    mu = s1_ref[...] * _INV_B
    var = s2_ref[...] * _INV_B - mu * mu
    hn = jnp.maximum(
        g1_ref[...] * (h1_ref[...] - mu) * lax.rsqrt(var + EPS) + be1_ref[...],
        0.0)
    h2 = jnp.dot(hn, w2_ref[...], preferred_element_type=_F32) + b2_ref[...]
    h2_ref[...] = h2
    t1 = jnp.sum(h2, axis=0, keepdims=True)
    t2 = jnp.sum(h2 * h2, axis=0, keepdims=True)

    @pl.when(i == 0)
    def _():
        t1_ref[...] = t1
        t2_ref[...] = t2

    @pl.when(i > 0)
    def _():
        t1_ref[...] += t1
        t2_ref[...] += t2


_b_call = pl.pallas_call(
    _b_body,
    grid=(NB,),
    in_specs=[
        pl.BlockSpec((BS, H1), lambda i: (i, 0)),
        pl.BlockSpec((1, H1), lambda i: (0, 0)),
        pl.BlockSpec((1, H1), lambda i: (0, 0)),
        pl.BlockSpec((H1,), lambda i: (0,)),
        pl.BlockSpec((H1,), lambda i: (0,)),
        pl.BlockSpec((H1, H2), lambda i: (0, 0)),
        pl.BlockSpec((H2,), lambda i: (0,)),
    ],
    out_specs=[
        pl.BlockSpec((BS, H2), lambda i: (i, 0)),
        pl.BlockSpec((1, H2), lambda i: (0, 0)),
        pl.BlockSpec((1, H2), lambda i: (0, 0)),
    ],
    out_shape=[
        jax.ShapeDtypeStruct((B, H2), _F32),
        jax.ShapeDtypeStruct((1, H2), _F32),
        jax.ShapeDtypeStruct((1, H2), _F32),
    ],
)


def _c_body(h2_ref, t1_ref, t2_ref, g2_ref, be2_ref, w3_ref, wide_ref,
            b3w_ref, out_ref):
    mu = t1_ref[...] * _INV_B
    var = t2_ref[...] * _INV_B - mu * mu
    hn = jnp.maximum(
        g2_ref[...] * (h2_ref[...] - mu) * lax.rsqrt(var + EPS) + be2_ref[...],
        0.0)
    out_ref[...] = (jnp.sum(hn * w3_ref[...], axis=1) + wide_ref[...]
                    + b3w_ref[0, 0])


_c_call = pl.pallas_call(
    _c_body,
    grid=(NB,),
    in_specs=[
        pl.BlockSpec((BS, H2), lambda i: (i, 0)),
        pl.BlockSpec((1, H2), lambda i: (0, 0)),
        pl.BlockSpec((1, H2), lambda i: (0, 0)),
        pl.BlockSpec((H2,), lambda i: (0,)),
        pl.BlockSpec((H2,), lambda i: (0,)),
        pl.BlockSpec((1, H2), lambda i: (0, 0)),
        pl.BlockSpec((BS,), lambda i: (i,)),
        pl.BlockSpec(memory_space=pltpu.SMEM),
    ],
    out_specs=pl.BlockSpec((BS,), lambda i: (i,)),
    out_shape=jax.ShapeDtypeStruct((B,), _F32),
)


def kernel(num_x, cat_x, tables, W1, b1, g1, be1, W2, b2, g2, be2, W3, b3,
           Ww, bw):
    tab = tables.reshape(TAB_R, 128)
    idx = (cat_x.astype(jnp.int32)
           + (jnp.arange(NUM_CAT, dtype=jnp.int32) * VOCAB)[None, :]
           ).reshape(NW, GPW, 128)
    em = _make_sc_gather()(tab, idx)
    w1a, w1b = W1[:NUM_NUMERIC], W1[NUM_NUMERIC:]
    w1bp = jnp.pad(w1b, ((0, XD - NUM_CAT * EMB), (0, 0)))
    wwa = Ww[:NUM_NUMERIC, 0][None, :]
    wwb = jnp.pad(Ww[NUM_NUMERIC:, 0][None, :],
                  ((0, 0), (0, XD - NUM_CAT * EMB)))
    w3row = W3[:, 0][None, :]
    b3w = (b3 + bw).reshape(1, 1)
    h1, wide, s1, s2 = _a_call(num_x, em, w1a, w1bp, b1, wwa, wwb)
    h2, t1, t2 = _b_call(h1, s1, s2, g1, be1, W2, b2)
    return _c_call(h2, t1, t2, g2, be2, w3row, wide, b3w)


# speedup vs baseline: 1.0339x; 1.0339x over previous
import functools

import jax
import jax.numpy as jnp
from jax import lax
from jax.experimental import pallas as pl
from jax.experimental.pallas import tpu as pltpu
from jax.experimental.pallas import tpu_sc as plsc

B = 16384
NUM_NUMERIC = 13
NUM_CAT = 26
VOCAB = 100000
EMB = 16
H1 = 256
H2 = 128
EPS = 1e-5

TOT = B * NUM_CAT
NC, NS = 2, 16
NW = NC * NS
LPW = TOT // NW
GPW = LPW // 128
SPW = B // NW
FLUSH = 13
ROWS_F = FLUSH * 128 // NUM_CAT
XD = 512
VPF = 12504
TAB_R = NUM_CAT * VPF


FT = VOCAB // 128
FU = NUM_CAT * FT
TAIL0 = FT * 128


@functools.cache
def _make_sc_format():
    mesh = plsc.VectorSubcoreMesh(core_axis_name="c", subcore_axis_name="s")

    @functools.partial(
        pl.kernel,
        out_type=jax.ShapeDtypeStruct((TAB_R, 128), jnp.float32),
        mesh=mesh,
        scratch_types=[
            pltpu.VMEM((2, 16, 128), jnp.float32),
            pltpu.VMEM((2, 16, 128), jnp.float32),
            pltpu.SemaphoreType.DMA((2,)),
            pltpu.SemaphoreType.DMA((2,)),
        ],
        compiler_params=pltpu.CompilerParams(needs_layout_passes=False),
    )
    def _sc_format(tabt_hbm, tailc_hbm, out_hbm, in_v, tr_v, isems, osems):
        wid = lax.axis_index("s") * NC + lax.axis_index("c")
        iota = jnp.arange(16, dtype=jnp.int32)
        scols = [(dv << 4) + iota for dv in range(8)]
        n_units = (FU - wid + NW - 1) // NW

        def unit_uid(k):
            return wid + k * NW

        def in_src(uid):
            f = uid // FT
            t = lax.rem(uid, FT)
            return tabt_hbm.at[pl.ds(pl.multiple_of(f * 16, 16), 16),
                               pl.ds(pl.multiple_of(t * 128, 128), 128)]

        def out_dst(uid):
            f = uid // FT
            t = lax.rem(uid, FT)
            return out_hbm.at[
                pl.ds(pl.multiple_of(f * VPF + t * 16, 8), 16), :]

        def fire_in(k):
            slot = lax.rem(k, 2)
            return pltpu.async_copy(in_src(unit_uid(k)), in_v.at[slot],
                                    isems.at[slot])

        fire_in(0)

        def unit_body(k, carry):
            slot = lax.rem(k, 2)
            uid = unit_uid(k)

            @pl.when(k + 1 < n_units)
            def _():
                fire_in(k + 1)

            pltpu.make_async_copy(in_src(uid), in_v.at[slot],
                                  isems.at[slot]).wait()

            @pl.when(k >= 2)
            def _():
                pltpu.make_async_copy(tr_v.at[slot], out_dst(unit_uid(k - 2)),
                                      osems.at[slot]).wait()

            slotv = jnp.full((16,), slot, jnp.int32)

            def vb_body(vb, carry2):
                vbase = jnp.full((16,), vb * 8, jnp.int32)
                rowv = jnp.full((16,), vb, jnp.int32)
                for dv in range(8):
                    vals = plsc.load_gather(in_v, [slotv, iota, vbase + dv])
                    plsc.store_scatter(tr_v, [slotv, rowv, scols[dv]], vals)
                return carry2

            lax.fori_loop(0, 16, vb_body, 0)
            pltpu.async_copy(tr_v.at[slot], out_dst(uid), osems.at[slot])
            return carry

        lax.fori_loop(0, n_units, unit_body, 0)

        @pl.when(n_units >= 2)
        def _():
            pltpu.make_async_copy(tr_v.at[lax.rem(n_units - 2, 2)],
                                  out_dst(unit_uid(n_units - 2)),
                                  osems.at[lax.rem(n_units - 2, 2)]).wait()
        pltpu.make_async_copy(tr_v.at[lax.rem(n_units - 1, 2)],
                              out_dst(unit_uid(n_units - 1)),
                              osems.at[lax.rem(n_units - 1, 2)]).wait()

        @pl.when(wid < NUM_CAT)
        def _():
            f = wid
            pltpu.sync_copy(tailc_hbm.at[f], tr_v.at[0, pl.ds(0, 8), :])
            pltpu.sync_copy(
                tr_v.at[0, pl.ds(0, 8), :],
                out_hbm.at[
                    pl.ds(pl.multiple_of(f * VPF + TAIL0 // 8, 8), 8), :])

    return _sc_format


@functools.cache
def _make_sc_gather():
    mesh = plsc.VectorSubcoreMesh(core_axis_name="c", subcore_axis_name="s")

    @functools.partial(
        pl.kernel,
        out_type=jax.ShapeDtypeStruct((B, XD), jnp.float32),
        mesh=mesh,
        scratch_types=[
            pltpu.VMEM((GPW, 128), jnp.int32),
            pltpu.VMEM((GPW, 128), jnp.int32),
            pltpu.VMEM((2, 128, 128), jnp.float32),
            pltpu.VMEM((ROWS_F, XD), jnp.float32),
            pltpu.SemaphoreType.DMA((2,)),
        ],
        compiler_params=pltpu.CompilerParams(needs_layout_passes=False),
    )
    def _sc_gather(tab_hbm, idx_hbm, out_hbm, idx_v, idxr_v, rows_v, asm_v,
                   sems):
        wid = lax.axis_index("s") * NC + lax.axis_index("c")
        pltpu.sync_copy(idx_hbm.at[wid], idx_v)

        iota = jnp.arange(16, dtype=jnp.int32)
        zeros = jnp.zeros((16,), jnp.float32)

        def zero_body(r, carry):
            for c in range(EMB * NUM_CAT // 16, XD // 16):
                asm_v[r, pl.ds(c * 16, 16)] = zeros
            return carry

        lax.fori_loop(0, ROWS_F, zero_body, 0)

        def shift_body(g, carry):
            for c in range(8):
                idxr_v[g, pl.ds(c * 16, 16)] = (
                    idx_v[g, pl.ds(c * 16, 16)] >> 3)
            return carry

        lax.fori_loop(0, GPW, shift_body, 0)

        def fire(g, slot):
            return pltpu.async_copy(
                tab_hbm.at[idxr_v.at[g]], rows_v.at[slot], sems.at[slot])

        fire(0, 0)

        def group_body(g, carry):
            slot = lax.rem(g, 2)

            @pl.when(g < GPW - 1)
            def _():
                fire(g + 1, lax.rem(g + 1, 2))

            pltpu.make_async_copy(
                tab_hbm.at[idxr_v.at[g]], rows_v.at[slot],
                sems.at[slot]).wait()

            base = lax.rem(g, FLUSH) * 128
            arow_base = jnp.full((16,), 0, jnp.int32)
            slotv = jnp.full((16,), slot, jnp.int32)
            for c in range(8):
                flat = base + c * 16 + iota
                idxc = idx_v[g, pl.ds(c * 16, 16)]
                off = (idxc & 7) << 4
                srow = flat // NUM_CAT
                scol = lax.rem(flat, NUM_CAT) << 4
                jvec = jnp.full((16,), c * 16, jnp.int32) + iota
                for e in range(EMB):
                    vals = plsc.load_gather(rows_v, [slotv, jvec, off + e])
                    plsc.store_scatter(asm_v, [srow, scol + e], vals)

            @pl.when(lax.rem(g, FLUSH) == FLUSH - 1)
            def _():
                s = g // FLUSH
                pltpu.sync_copy(
                    asm_v,
                    out_hbm.at[pl.ds(wid * SPW + s * ROWS_F, ROWS_F)])

            return carry

        lax.fori_loop(0, GPW, group_body, 0)

    return _sc_gather


BS = 1024
NB = B // BS
_INV_B = 1.0 / B
_F32 = jnp.float32


def _a_body(nx_ref, em_ref, w1a_ref, w1b_ref, b1_ref, wwa_ref, wwb_ref,
            h1_ref, wide_ref, s1_ref, s2_ref):
    i = pl.program_id(0)
    nx = nx_ref[...]
    em = em_ref[...]
    h = (jnp.dot(nx, w1a_ref[...], preferred_element_type=_F32)
         + jnp.dot(em, w1b_ref[...], preferred_element_type=_F32)
         + b1_ref[...])
    h1_ref[...] = h
    wide_ref[...] = (jnp.sum(nx * wwa_ref[...], axis=1)
                     + jnp.sum(em * wwb_ref[...], axis=1))
    s1 = jnp.sum(h, axis=0, keepdims=True)
    s2 = jnp.sum(h * h, axis=0, keepdims=True)

    @pl.when(i == 0)
    def _():
        s1_ref[...] = s1
        s2_ref[...] = s2

    @pl.when(i > 0)
    def _():
        s1_ref[...] += s1
        s2_ref[...] += s2


_a_call = pl.pallas_call(
    _a_body,
    grid=(NB,),
    in_specs=[
        pl.BlockSpec((BS, NUM_NUMERIC), lambda i: (i, 0)),
        pl.BlockSpec((BS, XD), lambda i: (i, 0)),
        pl.BlockSpec((NUM_NUMERIC, H1), lambda i: (0, 0)),
        pl.BlockSpec((XD, H1), lambda i: (0, 0)),
        pl.BlockSpec((H1,), lambda i: (0,)),
        pl.BlockSpec((1, NUM_NUMERIC), lambda i: (0, 0)),
        pl.BlockSpec((1, XD), lambda i: (0, 0)),
    ],
    out_specs=[
        pl.BlockSpec((BS, H1), lambda i: (i, 0)),
        pl.BlockSpec((BS,), lambda i: (i,)),
        pl.BlockSpec((1, H1), lambda i: (0, 0)),
        pl.BlockSpec((1, H1), lambda i: (0, 0)),
    ],
    out_shape=[
        jax.ShapeDtypeStruct((B, H1), _F32),
        jax.ShapeDtypeStruct((B,), _F32),
        jax.ShapeDtypeStruct((1, H1), _F32),
        jax.ShapeDtypeStruct((1, H1), _F32),
    ],
)


def _b_body(h1_ref, s1_ref, s2_ref, g1_ref, be1_ref, w2_ref, b2_ref,
            h2_ref, t1_ref, t2_ref):
    i = pl.program_id(0)
    mu = s1_ref[...] * _INV_B
    var = s2_ref[...] * _INV_B - mu * mu
    hn = jnp.maximum(
        g1_ref[...] * (h1_ref[...] - mu) * lax.rsqrt(var + EPS) + be1_ref[...],
        0.0)
    h2 = jnp.dot(hn, w2_ref[...], preferred_element_type=_F32) + b2_ref[...]
    h2_ref[...] = h2
    t1 = jnp.sum(h2, axis=0, keepdims=True)
    t2 = jnp.sum(h2 * h2, axis=0, keepdims=True)

    @pl.when(i == 0)
    def _():
        t1_ref[...] = t1
        t2_ref[...] = t2

    @pl.when(i > 0)
    def _():
        t1_ref[...] += t1
        t2_ref[...] += t2


_b_call = pl.pallas_call(
    _b_body,
    grid=(NB,),
    in_specs=[
        pl.BlockSpec((BS, H1), lambda i: (i, 0)),
        pl.BlockSpec((1, H1), lambda i: (0, 0)),
        pl.BlockSpec((1, H1), lambda i: (0, 0)),
        pl.BlockSpec((H1,), lambda i: (0,)),
        pl.BlockSpec((H1,), lambda i: (0,)),
        pl.BlockSpec((H1, H2), lambda i: (0, 0)),
        pl.BlockSpec((H2,), lambda i: (0,)),
    ],
    out_specs=[
        pl.BlockSpec((BS, H2), lambda i: (i, 0)),
        pl.BlockSpec((1, H2), lambda i: (0, 0)),
        pl.BlockSpec((1, H2), lambda i: (0, 0)),
    ],
    out_shape=[
        jax.ShapeDtypeStruct((B, H2), _F32),
        jax.ShapeDtypeStruct((1, H2), _F32),
        jax.ShapeDtypeStruct((1, H2), _F32),
    ],
)


def _c_body(h2_ref, t1_ref, t2_ref, g2_ref, be2_ref, w3_ref, wide_ref,
            b3w_ref, out_ref):
    mu = t1_ref[...] * _INV_B
    var = t2_ref[...] * _INV_B - mu * mu
    hn = jnp.maximum(
        g2_ref[...] * (h2_ref[...] - mu) * lax.rsqrt(var + EPS) + be2_ref[...],
        0.0)
    out_ref[...] = (jnp.sum(hn * w3_ref[...], axis=1) + wide_ref[...]
                    + b3w_ref[0, 0])


_c_call = pl.pallas_call(
    _c_body,
    grid=(NB,),
    in_specs=[
        pl.BlockSpec((BS, H2), lambda i: (i, 0)),
        pl.BlockSpec((1, H2), lambda i: (0, 0)),
        pl.BlockSpec((1, H2), lambda i: (0, 0)),
        pl.BlockSpec((H2,), lambda i: (0,)),
        pl.BlockSpec((H2,), lambda i: (0,)),
        pl.BlockSpec((1, H2), lambda i: (0, 0)),
        pl.BlockSpec((BS,), lambda i: (i,)),
        pl.BlockSpec(memory_space=pltpu.SMEM),
    ],
    out_specs=pl.BlockSpec((BS,), lambda i: (i,)),
    out_shape=jax.ShapeDtypeStruct((B,), _F32),
)


def kernel(num_x, cat_x, tables, W1, b1, g1, be1, W2, b2, g2, be2, W3, b3,
           Ww, bw):
    tabt = tables.transpose(0, 2, 1).reshape(NUM_CAT * EMB, VOCAB)
    tailc = jnp.pad(tables[:, TAIL0:, :].reshape(NUM_CAT, 4, 128),
                    ((0, 0), (0, 4), (0, 0)))
    tab = _make_sc_format()(tabt, tailc)
    idx = (cat_x.astype(jnp.int32)
           + (jnp.arange(NUM_CAT, dtype=jnp.int32) * (VPF * 8))[None, :]
           ).reshape(NW, GPW, 128)
    em = _make_sc_gather()(tab, idx)
    w1a, w1b = W1[:NUM_NUMERIC], W1[NUM_NUMERIC:]
    w1bp = jnp.pad(w1b, ((0, XD - NUM_CAT * EMB), (0, 0)))
    wwa = Ww[:NUM_NUMERIC, 0][None, :]
    wwb = jnp.pad(Ww[NUM_NUMERIC:, 0][None, :],
                  ((0, 0), (0, XD - NUM_CAT * EMB)))
    w3row = W3[:, 0][None, :]
    b3w = (b3 + bw).reshape(1, 1)
    h1, wide, s1, s2 = _a_call(num_x, em, w1a, w1bp, b1, wwa, wwb)
    h2, t1, t2 = _b_call(h1, s1, s2, g1, be1, W2, b2)
    return _c_call(h2, t1, t2, g2, be2, w3row, wide, b3w)


# speedup vs baseline: 1.0411x; 1.0069x over previous
import functools

import jax
import jax.numpy as jnp
from jax import lax
from jax.experimental import pallas as pl
from jax.experimental.pallas import tpu as pltpu
from jax.experimental.pallas import tpu_sc as plsc

B = 16384
NUM_NUMERIC = 13
NUM_CAT = 26
VOCAB = 100000
EMB = 16
H1 = 256
H2 = 128
EPS = 1e-5

TOT = B * NUM_CAT
NC, NS = 2, 16
NW = NC * NS
LPW = TOT // NW
GPW = LPW // 128
SPW = B // NW
FLUSH = 13
ROWS_F = FLUSH * 128 // NUM_CAT
XD = 512
VPF = 12504
TAB_R = NUM_CAT * VPF


UB = 512
NBU = VOCAB // UB
BIGU = NUM_CAT * NBU
SMALL0 = NBU * UB
TAIL0 = SMALL0 + 128
NRING = 4


@functools.cache
def _make_sc_format():
    mesh = plsc.VectorSubcoreMesh(core_axis_name="c", subcore_axis_name="s")

    @functools.partial(
        pl.kernel,
        out_type=jax.ShapeDtypeStruct((TAB_R, 128), jnp.float32),
        mesh=mesh,
        scratch_types=[
            pltpu.VMEM((NRING, 16, UB), jnp.float32),
            pltpu.VMEM((NRING, UB // 8, 128), jnp.float32),
            pltpu.VMEM((16, 128), jnp.float32),
            pltpu.SemaphoreType.DMA((NRING,)),
            pltpu.SemaphoreType.DMA((NRING,)),
            pltpu.SemaphoreType.DMA,
        ],
        compiler_params=pltpu.CompilerParams(needs_layout_passes=False),
    )
    def _sc_format(tabt_hbm, tailc_hbm, out_hbm, in_v, tr_v, sm_v, isems,
                   osems, ssem):
        wid = lax.axis_index("s") * NC + lax.axis_index("c")
        iota = jnp.arange(16, dtype=jnp.int32)
        scols = [(dv << 4) + iota for dv in range(8)]
        n_units = (BIGU - wid + NW - 1) // NW

        def unit_uid(k):
            return wid + k * NW

        def in_src(uid):
            f = uid // NBU
            t = lax.rem(uid, NBU)
            return tabt_hbm.at[pl.ds(pl.multiple_of(f * 16, 16), 16),
                               pl.ds(pl.multiple_of(t * UB, UB), UB)]

        def out_dst(uid):
            f = uid // NBU
            t = lax.rem(uid, NBU)
            return out_hbm.at[
                pl.ds(pl.multiple_of(f * VPF + t * (UB // 8), 8), UB // 8), :]

        def fire_in(k):
            slot = lax.rem(k, NRING)
            return pltpu.async_copy(in_src(unit_uid(k)), in_v.at[slot],
                                    isems.at[slot])

        for p in range(NRING - 1):
            fire_in(p)

        def unit_body(k, carry):
            slot = lax.rem(k, NRING)
            uid = unit_uid(k)

            @pl.when(k + NRING - 1 < n_units)
            def _():
                fire_in(k + NRING - 1)

            pltpu.make_async_copy(in_src(uid), in_v.at[slot],
                                  isems.at[slot]).wait()

            @pl.when(k >= NRING)
            def _():
                pltpu.make_async_copy(tr_v.at[slot],
                                      out_dst(unit_uid(k - NRING)),
                                      osems.at[slot]).wait()

            slotv = jnp.full((16,), slot, jnp.int32)

            def vb_body(vb, carry2):
                vbase = jnp.full((16,), vb * 8, jnp.int32)
                rowv = jnp.full((16,), vb, jnp.int32)
                for dv in range(8):
                    vals = plsc.load_gather(in_v, [slotv, iota, vbase + dv])
                    plsc.store_scatter(tr_v, [slotv, rowv, scols[dv]], vals)
                return carry2

            lax.fori_loop(0, UB // 8, vb_body, 0)
            pltpu.async_copy(tr_v.at[slot], out_dst(uid), osems.at[slot])
            return carry

        lax.fori_loop(0, n_units, unit_body, 0)

        def drain_body(j, carry):
            pltpu.make_async_copy(tr_v.at[lax.rem(j, NRING)],
                                  out_dst(unit_uid(j)),
                                  osems.at[lax.rem(j, NRING)]).wait()
            return carry

        lax.fori_loop(lax.max(n_units - NRING, 0), n_units, drain_body, 0)

        @pl.when(wid < NUM_CAT)
        def _():
            f = wid
            pltpu.sync_copy(
                tabt_hbm.at[pl.ds(pl.multiple_of(f * 16, 16), 16),
                            pl.ds(SMALL0, 128)],
                sm_v)
            zv = jnp.zeros((16,), jnp.int32)

            def vb_body(vb, carry2):
                vbase = jnp.full((16,), vb * 8, jnp.int32)
                rowv = jnp.full((16,), vb, jnp.int32)
                for dv in range(8):
                    vals = plsc.load_gather(sm_v, [iota, vbase + dv])
                    plsc.store_scatter(tr_v, [zv, rowv, scols[dv]], vals)
                return carry2

            lax.fori_loop(0, 16, vb_body, 0)
            pltpu.sync_copy(
                tr_v.at[0, pl.ds(0, 16), :],
                out_hbm.at[
                    pl.ds(pl.multiple_of(f * VPF + SMALL0 // 8, 8), 16), :])
            pltpu.sync_copy(tailc_hbm.at[f], sm_v.at[pl.ds(0, 8), :])
            pltpu.sync_copy(
                sm_v.at[pl.ds(0, 8), :],
                out_hbm.at[
                    pl.ds(pl.multiple_of(f * VPF + TAIL0 // 8, 8), 8), :])

    return _sc_format


@functools.cache
def _make_sc_gather():
    mesh = plsc.VectorSubcoreMesh(core_axis_name="c", subcore_axis_name="s")

    @functools.partial(
        pl.kernel,
        out_type=jax.ShapeDtypeStruct((B, XD), jnp.float32),
        mesh=mesh,
        scratch_types=[
            pltpu.VMEM((GPW, 128), jnp.int32),
            pltpu.VMEM((GPW, 128), jnp.int32),
            pltpu.VMEM((4, 128, 128), jnp.float32),
            pltpu.VMEM((ROWS_F, XD), jnp.float32),
            pltpu.SemaphoreType.DMA((4,)),
        ],
        compiler_params=pltpu.CompilerParams(needs_layout_passes=False),
    )
    def _sc_gather(tab_hbm, idx_hbm, out_hbm, idx_v, idxr_v, rows_v, asm_v,
                   sems):
        wid = lax.axis_index("s") * NC + lax.axis_index("c")
        pltpu.sync_copy(idx_hbm.at[wid], idx_v)

        iota = jnp.arange(16, dtype=jnp.int32)
        zeros = jnp.zeros((16,), jnp.float32)

        def zero_body(r, carry):
            for c in range(EMB * NUM_CAT // 16, XD // 16):
                asm_v[r, pl.ds(c * 16, 16)] = zeros
            return carry

        lax.fori_loop(0, ROWS_F, zero_body, 0)

        def shift_body(g, carry):
            for c in range(8):
                idxr_v[g, pl.ds(c * 16, 16)] = (
                    idx_v[g, pl.ds(c * 16, 16)] >> 3)
            return carry

        lax.fori_loop(0, GPW, shift_body, 0)

        def fire(g, slot):
            return pltpu.async_copy(
                tab_hbm.at[idxr_v.at[g]], rows_v.at[slot], sems.at[slot])

        for p in range(3):
            fire(p, p)

        def group_body(g, carry):
            slot = lax.rem(g, 4)

            @pl.when(g + 3 < GPW)
            def _():
                fire(g + 3, lax.rem(g + 3, 4))

            pltpu.make_async_copy(
                tab_hbm.at[idxr_v.at[g]], rows_v.at[slot],
                sems.at[slot]).wait()

            base = lax.rem(g, FLUSH) * 128
            arow_base = jnp.full((16,), 0, jnp.int32)
            slotv = jnp.full((16,), slot, jnp.int32)
            for c in range(8):
                flat = base + c * 16 + iota
                idxc = idx_v[g, pl.ds(c * 16, 16)]
                off = (idxc & 7) << 4
                srow = flat // NUM_CAT
                scol = lax.rem(flat, NUM_CAT) << 4
                jvec = jnp.full((16,), c * 16, jnp.int32) + iota
                for e in range(EMB):
                    vals = plsc.load_gather(rows_v, [slotv, jvec, off + e])
                    plsc.store_scatter(asm_v, [srow, scol + e], vals)

            @pl.when(lax.rem(g, FLUSH) == FLUSH - 1)
            def _():
                s = g // FLUSH
                pltpu.sync_copy(
                    asm_v,
                    out_hbm.at[pl.ds(wid * SPW + s * ROWS_F, ROWS_F)])

            return carry

        lax.fori_loop(0, GPW, group_body, 0)

    return _sc_gather


BS = 1024
NB = B // BS
_INV_B = 1.0 / B
_F32 = jnp.float32


def _a_body(nx_ref, em_ref, w1a_ref, w1b_ref, b1_ref, wwa_ref, wwb_ref,
            h1_ref, wide_ref, s1_ref, s2_ref):
    i = pl.program_id(0)
    nx = nx_ref[...]
    em = em_ref[...]
    h = (jnp.dot(nx, w1a_ref[...], preferred_element_type=_F32)
         + jnp.dot(em, w1b_ref[...], preferred_element_type=_F32)
         + b1_ref[...])
    h1_ref[...] = h
    wide_ref[...] = (jnp.sum(nx * wwa_ref[...], axis=1)
                     + jnp.sum(em * wwb_ref[...], axis=1))
    s1 = jnp.sum(h, axis=0, keepdims=True)
    s2 = jnp.sum(h * h, axis=0, keepdims=True)

    @pl.when(i == 0)
    def _():
        s1_ref[...] = s1
        s2_ref[...] = s2

    @pl.when(i > 0)
    def _():
        s1_ref[...] += s1
        s2_ref[...] += s2


_a_call = pl.pallas_call(
    _a_body,
    grid=(NB,),
    in_specs=[
        pl.BlockSpec((BS, NUM_NUMERIC), lambda i: (i, 0)),
        pl.BlockSpec((BS, XD), lambda i: (i, 0)),
        pl.BlockSpec((NUM_NUMERIC, H1), lambda i: (0, 0)),
        pl.BlockSpec((XD, H1), lambda i: (0, 0)),
        pl.BlockSpec((H1,), lambda i: (0,)),
        pl.BlockSpec((1, NUM_NUMERIC), lambda i: (0, 0)),
        pl.BlockSpec((1, XD), lambda i: (0, 0)),
    ],
    out_specs=[
        pl.BlockSpec((BS, H1), lambda i: (i, 0)),
        pl.BlockSpec((BS,), lambda i: (i,)),
        pl.BlockSpec((1, H1), lambda i: (0, 0)),
        pl.BlockSpec((1, H1), lambda i: (0, 0)),
    ],
    out_shape=[
        jax.ShapeDtypeStruct((B, H1), _F32),
        jax.ShapeDtypeStruct((B,), _F32),
        jax.ShapeDtypeStruct((1, H1), _F32),
        jax.ShapeDtypeStruct((1, H1), _F32),
    ],
)


def _b_body(h1_ref, s1_ref, s2_ref, g1_ref, be1_ref, w2_ref, b2_ref,
            h2_ref, t1_ref, t2_ref):
    i = pl.program_id(0)
    mu = s1_ref[...] * _INV_B
    var = s2_ref[...] * _INV_B - mu * mu
    hn = jnp.maximum(
        g1_ref[...] * (h1_ref[...] - mu) * lax.rsqrt(var + EPS) + be1_ref[...],
        0.0)
    h2 = jnp.dot(hn, w2_ref[...], preferred_element_type=_F32) + b2_ref[...]
    h2_ref[...] = h2
    t1 = jnp.sum(h2, axis=0, keepdims=True)
    t2 = jnp.sum(h2 * h2, axis=0, keepdims=True)

    @pl.when(i == 0)
    def _():
        t1_ref[...] = t1
        t2_ref[...] = t2

    @pl.when(i > 0)
    def _():
        t1_ref[...] += t1
        t2_ref[...] += t2


_b_call = pl.pallas_call(
    _b_body,
    grid=(NB,),
    in_specs=[
        pl.BlockSpec((BS, H1), lambda i: (i, 0)),
        pl.BlockSpec((1, H1), lambda i: (0, 0)),
        pl.BlockSpec((1, H1), lambda i: (0, 0)),
        pl.BlockSpec((H1,), lambda i: (0,)),
        pl.BlockSpec((H1,), lambda i: (0,)),
        pl.BlockSpec((H1, H2), lambda i: (0, 0)),
        pl.BlockSpec((H2,), lambda i: (0,)),
    ],
    out_specs=[
        pl.BlockSpec((BS, H2), lambda i: (i, 0)),
        pl.BlockSpec((1, H2), lambda i: (0, 0)),
        pl.BlockSpec((1, H2), lambda i: (0, 0)),
    ],
    out_shape=[
        jax.ShapeDtypeStruct((B, H2), _F32),
        jax.ShapeDtypeStruct((1, H2), _F32),
        jax.ShapeDtypeStruct((1, H2), _F32),
    ],
)


def _c_body(h2_ref, t1_ref, t2_ref, g2_ref, be2_ref, w3_ref, wide_ref,
            b3w_ref, out_ref):
    mu = t1_ref[...] * _INV_B
    var = t2_ref[...] * _INV_B - mu * mu
    hn = jnp.maximum(
        g2_ref[...] * (h2_ref[...] - mu) * lax.rsqrt(var + EPS) + be2_ref[...],
        0.0)
    out_ref[...] = (jnp.sum(hn * w3_ref[...], axis=1) + wide_ref[...]
                    + b3w_ref[0, 0])


_c_call = pl.pallas_call(
    _c_body,
    grid=(NB,),
    in_specs=[
        pl.BlockSpec((BS, H2), lambda i: (i, 0)),
        pl.BlockSpec((1, H2), lambda i: (0, 0)),
        pl.BlockSpec((1, H2), lambda i: (0, 0)),
        pl.BlockSpec((H2,), lambda i: (0,)),
        pl.BlockSpec((H2,), lambda i: (0,)),
        pl.BlockSpec((1, H2), lambda i: (0, 0)),
        pl.BlockSpec((BS,), lambda i: (i,)),
        pl.BlockSpec(memory_space=pltpu.SMEM),
    ],
    out_specs=pl.BlockSpec((BS,), lambda i: (i,)),
    out_shape=jax.ShapeDtypeStruct((B,), _F32),
)


def kernel(num_x, cat_x, tables, W1, b1, g1, be1, W2, b2, g2, be2, W3, b3,
           Ww, bw):
    tabt = tables.transpose(0, 2, 1).reshape(NUM_CAT * EMB, VOCAB)
    tailc = jnp.pad(tables[:, TAIL0:, :].reshape(NUM_CAT, 4, 128),
                    ((0, 0), (0, 4), (0, 0)))
    tab = _make_sc_format()(tabt, tailc)
    idx = (cat_x.astype(jnp.int32)
           + (jnp.arange(NUM_CAT, dtype=jnp.int32) * (VPF * 8))[None, :]
           ).reshape(NW, GPW, 128)
    em = _make_sc_gather()(tab, idx)
    w1a, w1b = W1[:NUM_NUMERIC], W1[NUM_NUMERIC:]
    w1bp = jnp.pad(w1b, ((0, XD - NUM_CAT * EMB), (0, 0)))
    wwa = Ww[:NUM_NUMERIC, 0][None, :]
    wwb = jnp.pad(Ww[NUM_NUMERIC:, 0][None, :],
                  ((0, 0), (0, XD - NUM_CAT * EMB)))
    w3row = W3[:, 0][None, :]
    b3w = (b3 + bw).reshape(1, 1)
    h1, wide, s1, s2 = _a_call(num_x, em, w1a, w1bp, b1, wwa, wwb)
    h2, t1, t2 = _b_call(h1, s1, s2, g1, be1, W2, b2)
    return _c_call(h2, t1, t2, g2, be2, w3row, wide, b3w)


# speedup vs baseline: 1.6555x; 1.5902x over previous
import functools

import jax
import jax.numpy as jnp
from jax import lax
from jax.experimental import pallas as pl
from jax.experimental.pallas import tpu as pltpu
from jax.experimental.pallas import tpu_sc as plsc

B = 16384
NUM_NUMERIC = 13
NUM_CAT = 26
VOCAB = 100000
EMB = 16
H1 = 256
H2 = 128
EPS = 1e-5

TOT = B * NUM_CAT
NC, NS = 2, 16
NW = NC * NS
LPW = TOT // NW
GPW = LPW // 128
SPW = B // NW
FLUSH = 13
ROWS_F = FLUSH * 128 // NUM_CAT
XD = 512
VPF = 12504
TAB_R = NUM_CAT * VPF


UB = 512
NBU = VOCAB // UB
BIGU = NUM_CAT * NBU
SMALL0 = NBU * UB
TAIL0 = SMALL0 + 128
NRING = 4


@functools.cache
def _make_sc_format():
    mesh = plsc.VectorSubcoreMesh(core_axis_name="c", subcore_axis_name="s")

    @functools.partial(
        pl.kernel,
        out_type=jax.ShapeDtypeStruct((TAB_R, 128), jnp.float32),
        mesh=mesh,
        scratch_types=[
            pltpu.VMEM((NRING, 16, UB), jnp.float32),
            pltpu.VMEM((NRING, UB // 8, 128), jnp.float32),
            pltpu.VMEM((16, 128), jnp.float32),
            pltpu.SemaphoreType.DMA((NRING,)),
            pltpu.SemaphoreType.DMA((NRING,)),
            pltpu.SemaphoreType.DMA,
        ],
        compiler_params=pltpu.CompilerParams(needs_layout_passes=False),
    )
    def _sc_format(tabt_hbm, tailc_hbm, out_hbm, in_v, tr_v, sm_v, isems,
                   osems, ssem):
        wid = lax.axis_index("s") * NC + lax.axis_index("c")
        iota = jnp.arange(16, dtype=jnp.int32)
        n_units = (BIGU - wid + NW - 1) // NW

        def unit_uid(k):
            return wid + k * NW

        def in_src(uid):
            f = uid // NBU
            t = lax.rem(uid, NBU)
            return tabt_hbm.at[pl.ds(pl.multiple_of(f * 16, 16), 16),
                               pl.ds(pl.multiple_of(t * UB, UB), UB)]

        def out_dst(uid):
            f = uid // NBU
            t = lax.rem(uid, NBU)
            return out_hbm.at[
                pl.ds(pl.multiple_of(f * VPF + t * (UB // 8), 8), UB // 8), :]

        def fire_in(k):
            slot = lax.rem(k, NRING)
            return pltpu.async_copy(in_src(unit_uid(k)), in_v.at[slot],
                                    isems.at[slot])

        for p in range(NRING - 1):
            fire_in(p)

        def unit_body(k, carry):
            slot = lax.rem(k, NRING)
            uid = unit_uid(k)

            @pl.when(k + NRING - 1 < n_units)
            def _():
                fire_in(k + NRING - 1)

            pltpu.make_async_copy(in_src(uid), in_v.at[slot],
                                  isems.at[slot]).wait()

            @pl.when(k >= NRING)
            def _():
                pltpu.make_async_copy(tr_v.at[slot],
                                      out_dst(unit_uid(k - NRING)),
                                      osems.at[slot]).wait()

            slotv = jnp.full((16,), slot, jnp.int32)

            def vb_body(vb, carry2):
                vbase = jnp.full((16,), vb * 8, jnp.int32)
                vals = [plsc.load_gather(in_v, [slotv, iota, vbase + dv])
                        for dv in range(8)]
                for dv in range(8):
                    tr_v[slot, vb, pl.ds(dv * 16, 16)] = vals[dv]
                return carry2

            lax.fori_loop(0, UB // 8, vb_body, 0)
            pltpu.async_copy(tr_v.at[slot], out_dst(uid), osems.at[slot])
            return carry

        lax.fori_loop(0, n_units, unit_body, 0)

        def drain_body(j, carry):
            pltpu.make_async_copy(tr_v.at[lax.rem(j, NRING)],
                                  out_dst(unit_uid(j)),
                                  osems.at[lax.rem(j, NRING)]).wait()
            return carry

        lax.fori_loop(lax.max(n_units - NRING, 0), n_units, drain_body, 0)

        @pl.when(wid < NUM_CAT)
        def _():
            f = wid
            pltpu.sync_copy(
                tabt_hbm.at[pl.ds(pl.multiple_of(f * 16, 16), 16),
                            pl.ds(SMALL0, 128)],
                sm_v)
            def vb_body(vb, carry2):
                vbase = jnp.full((16,), vb * 8, jnp.int32)
                vals = [plsc.load_gather(sm_v, [iota, vbase + dv])
                        for dv in range(8)]
                for dv in range(8):
                    tr_v[0, vb, pl.ds(dv * 16, 16)] = vals[dv]
                return carry2

            lax.fori_loop(0, 16, vb_body, 0)
            pltpu.sync_copy(
                tr_v.at[0, pl.ds(0, 16), :],
                out_hbm.at[
                    pl.ds(pl.multiple_of(f * VPF + SMALL0 // 8, 8), 16), :])
            pltpu.sync_copy(tailc_hbm.at[f], sm_v.at[pl.ds(0, 8), :])
            pltpu.sync_copy(
                sm_v.at[pl.ds(0, 8), :],
                out_hbm.at[
                    pl.ds(pl.multiple_of(f * VPF + TAIL0 // 8, 8), 8), :])

    return _sc_format


@functools.cache
def _make_sc_gather():
    mesh = plsc.VectorSubcoreMesh(core_axis_name="c", subcore_axis_name="s")

    @functools.partial(
        pl.kernel,
        out_type=jax.ShapeDtypeStruct((B, XD), jnp.float32),
        mesh=mesh,
        scratch_types=[
            pltpu.VMEM((GPW, 128), jnp.int32),
            pltpu.VMEM((GPW, 128), jnp.int32),
            pltpu.VMEM((4, 128, 128), jnp.float32),
            pltpu.VMEM((ROWS_F, XD), jnp.float32),
            pltpu.SemaphoreType.DMA((4,)),
        ],
        compiler_params=pltpu.CompilerParams(needs_layout_passes=False),
    )
    def _sc_gather(tab_hbm, idx_hbm, out_hbm, idx_v, idxr_v, rows_v, asm_v,
                   sems):
        wid = lax.axis_index("s") * NC + lax.axis_index("c")
        pltpu.sync_copy(idx_hbm.at[wid], idx_v)

        iota = jnp.arange(16, dtype=jnp.int32)
        zeros = jnp.zeros((16,), jnp.float32)

        def zero_body(r, carry):
            for c in range(EMB * NUM_CAT // 16, XD // 16):
                asm_v[r, pl.ds(c * 16, 16)] = zeros
            return carry

        lax.fori_loop(0, ROWS_F, zero_body, 0)

        def shift_body(g, carry):
            for c in range(8):
                idxr_v[g, pl.ds(c * 16, 16)] = (
                    idx_v[g, pl.ds(c * 16, 16)] >> 3)
            return carry

        lax.fori_loop(0, GPW, shift_body, 0)

        def fire(g, slot):
            return pltpu.async_copy(
                tab_hbm.at[idxr_v.at[g]], rows_v.at[slot], sems.at[slot])

        for p in range(3):
            fire(p, p)

        def group_body(g, carry):
            slot = lax.rem(g, 4)

            @pl.when(g + 3 < GPW)
            def _():
                fire(g + 3, lax.rem(g + 3, 4))

            pltpu.make_async_copy(
                tab_hbm.at[idxr_v.at[g]], rows_v.at[slot],
                sems.at[slot]).wait()

            base = lax.rem(g, FLUSH) * 128
            arow_base = jnp.full((16,), 0, jnp.int32)
            slotv = jnp.full((16,), slot, jnp.int32)
            for c in range(8):
                flat = base + c * 16 + iota
                idxc = idx_v[g, pl.ds(c * 16, 16)]
                off = (idxc & 7) << 4
                srow = flat // NUM_CAT
                scol = lax.rem(flat, NUM_CAT) << 4
                jvec = jnp.full((16,), c * 16, jnp.int32) + iota
                vals = [plsc.load_gather(rows_v, [slotv, jvec, off + e])
                        for e in range(EMB)]
                for e in range(EMB):
                    plsc.store_scatter(asm_v, [srow, scol + e], vals[e])

            @pl.when(lax.rem(g, FLUSH) == FLUSH - 1)
            def _():
                s = g // FLUSH
                pltpu.sync_copy(
                    asm_v,
                    out_hbm.at[pl.ds(wid * SPW + s * ROWS_F, ROWS_F)])

            return carry

        lax.fori_loop(0, GPW, group_body, 0)

    return _sc_gather


BS = 1024
NB = B // BS
_INV_B = 1.0 / B
_F32 = jnp.float32


def _a_body(nx_ref, em_ref, w1a_ref, w1b_ref, b1_ref, wwa_ref, wwb_ref,
            h1_ref, wide_ref, s1_ref, s2_ref):
    i = pl.program_id(0)
    nx = nx_ref[...]
    em = em_ref[...]
    h = (jnp.dot(nx, w1a_ref[...], preferred_element_type=_F32)
         + jnp.dot(em, w1b_ref[...], preferred_element_type=_F32)
         + b1_ref[...])
    h1_ref[...] = h
    wide_ref[...] = (jnp.sum(nx * wwa_ref[...], axis=1)
                     + jnp.sum(em * wwb_ref[...], axis=1))
    s1 = jnp.sum(h, axis=0, keepdims=True)
    s2 = jnp.sum(h * h, axis=0, keepdims=True)

    @pl.when(i == 0)
    def _():
        s1_ref[...] = s1
        s2_ref[...] = s2

    @pl.when(i > 0)
    def _():
        s1_ref[...] += s1
        s2_ref[...] += s2


_a_call = pl.pallas_call(
    _a_body,
    grid=(NB,),
    in_specs=[
        pl.BlockSpec((BS, NUM_NUMERIC), lambda i: (i, 0)),
        pl.BlockSpec((BS, XD), lambda i: (i, 0)),
        pl.BlockSpec((NUM_NUMERIC, H1), lambda i: (0, 0)),
        pl.BlockSpec((XD, H1), lambda i: (0, 0)),
        pl.BlockSpec((H1,), lambda i: (0,)),
        pl.BlockSpec((1, NUM_NUMERIC), lambda i: (0, 0)),
        pl.BlockSpec((1, XD), lambda i: (0, 0)),
    ],
    out_specs=[
        pl.BlockSpec((BS, H1), lambda i: (i, 0)),
        pl.BlockSpec((BS,), lambda i: (i,)),
        pl.BlockSpec((1, H1), lambda i: (0, 0)),
        pl.BlockSpec((1, H1), lambda i: (0, 0)),
    ],
    out_shape=[
        jax.ShapeDtypeStruct((B, H1), _F32),
        jax.ShapeDtypeStruct((B,), _F32),
        jax.ShapeDtypeStruct((1, H1), _F32),
        jax.ShapeDtypeStruct((1, H1), _F32),
    ],
)


def _b_body(h1_ref, s1_ref, s2_ref, g1_ref, be1_ref, w2_ref, b2_ref,
            h2_ref, t1_ref, t2_ref):
    i = pl.program_id(0)
    mu = s1_ref[...] * _INV_B
    var = s2_ref[...] * _INV_B - mu * mu
    hn = jnp.maximum(
        g1_ref[...] * (h1_ref[...] - mu) * lax.rsqrt(var + EPS) + be1_ref[...],
        0.0)
    h2 = jnp.dot(hn, w2_ref[...], preferred_element_type=_F32) + b2_ref[...]
    h2_ref[...] = h2
    t1 = jnp.sum(h2, axis=0, keepdims=True)
    t2 = jnp.sum(h2 * h2, axis=0, keepdims=True)

    @pl.when(i == 0)
    def _():
        t1_ref[...] = t1
        t2_ref[...] = t2

    @pl.when(i > 0)
    def _():
        t1_ref[...] += t1
        t2_ref[...] += t2


_b_call = pl.pallas_call(
    _b_body,
    grid=(NB,),
    in_specs=[
        pl.BlockSpec((BS, H1), lambda i: (i, 0)),
        pl.BlockSpec((1, H1), lambda i: (0, 0)),
        pl.BlockSpec((1, H1), lambda i: (0, 0)),
        pl.BlockSpec((H1,), lambda i: (0,)),
        pl.BlockSpec((H1,), lambda i: (0,)),
        pl.BlockSpec((H1, H2), lambda i: (0, 0)),
        pl.BlockSpec((H2,), lambda i: (0,)),
    ],
    out_specs=[
        pl.BlockSpec((BS, H2), lambda i: (i, 0)),
        pl.BlockSpec((1, H2), lambda i: (0, 0)),
        pl.BlockSpec((1, H2), lambda i: (0, 0)),
    ],
    out_shape=[
        jax.ShapeDtypeStruct((B, H2), _F32),
        jax.ShapeDtypeStruct((1, H2), _F32),
        jax.ShapeDtypeStruct((1, H2), _F32),
    ],
)


def _c_body(h2_ref, t1_ref, t2_ref, g2_ref, be2_ref, w3_ref, wide_ref,
            b3w_ref, out_ref):
    mu = t1_ref[...] * _INV_B
    var = t2_ref[...] * _INV_B - mu * mu
    hn = jnp.maximum(
        g2_ref[...] * (h2_ref[...] - mu) * lax.rsqrt(var + EPS) + be2_ref[...],
        0.0)
    out_ref[...] = (jnp.sum(hn * w3_ref[...], axis=1) + wide_ref[...]
                    + b3w_ref[0, 0])


_c_call = pl.pallas_call(
    _c_body,
    grid=(NB,),
    in_specs=[
        pl.BlockSpec((BS, H2), lambda i: (i, 0)),
        pl.BlockSpec((1, H2), lambda i: (0, 0)),
        pl.BlockSpec((1, H2), lambda i: (0, 0)),
        pl.BlockSpec((H2,), lambda i: (0,)),
        pl.BlockSpec((H2,), lambda i: (0,)),
        pl.BlockSpec((1, H2), lambda i: (0, 0)),
        pl.BlockSpec((BS,), lambda i: (i,)),
        pl.BlockSpec(memory_space=pltpu.SMEM),
    ],
    out_specs=pl.BlockSpec((BS,), lambda i: (i,)),
    out_shape=jax.ShapeDtypeStruct((B,), _F32),
)


def kernel(num_x, cat_x, tables, W1, b1, g1, be1, W2, b2, g2, be2, W3, b3,
           Ww, bw):
    tabt = tables.transpose(0, 2, 1).reshape(NUM_CAT * EMB, VOCAB)
    tailc = jnp.pad(tables[:, TAIL0:, :].reshape(NUM_CAT, 4, 128),
                    ((0, 0), (0, 4), (0, 0)))
    tab = _make_sc_format()(tabt, tailc)
    idx = (cat_x.astype(jnp.int32)
           + (jnp.arange(NUM_CAT, dtype=jnp.int32) * (VPF * 8))[None, :]
           ).reshape(NW, GPW, 128)
    em = _make_sc_gather()(tab, idx)
    w1a, w1b = W1[:NUM_NUMERIC], W1[NUM_NUMERIC:]
    w1bp = jnp.pad(w1b, ((0, XD - NUM_CAT * EMB), (0, 0)))
    wwa = Ww[:NUM_NUMERIC, 0][None, :]
    wwb = jnp.pad(Ww[NUM_NUMERIC:, 0][None, :],
                  ((0, 0), (0, XD - NUM_CAT * EMB)))
    w3row = W3[:, 0][None, :]
    b3w = (b3 + bw).reshape(1, 1)
    h1, wide, s1, s2 = _a_call(num_x, em, w1a, w1bp, b1, wwa, wwb)
    h2, t1, t2 = _b_call(h1, s1, s2, g1, be1, W2, b2)
    return _c_call(h2, t1, t2, g2, be2, w3row, wide, b3w)


# speedup vs baseline: 3.4154x; 2.0630x over previous
import functools

import jax
import jax.numpy as jnp
from jax import lax
from jax.experimental import pallas as pl
from jax.experimental.pallas import tpu as pltpu
from jax.experimental.pallas import tpu_sc as plsc

B = 16384
NUM_NUMERIC = 13
NUM_CAT = 26
VOCAB = 100000
EMB = 16
H1 = 256
H2 = 128
EPS = 1e-5

TOT = B * NUM_CAT
NC, NS = 2, 16
NW = NC * NS
LPW = TOT // NW
GPW = LPW // 128
SPW = B // NW
FLUSH = 13
ROWS_F = FLUSH * 128 // NUM_CAT
XD = 512
VPF = 12504
TAB_R = NUM_CAT * VPF


UB = 512
NBU = VOCAB // UB
BIGU = NUM_CAT * NBU
SMALL0 = NBU * UB
TAIL0 = SMALL0 + 128
NRING = 4


@functools.cache
def _make_sc_format():
    mesh = plsc.VectorSubcoreMesh(core_axis_name="c", subcore_axis_name="s")

    @functools.partial(
        pl.kernel,
        out_type=jax.ShapeDtypeStruct((TAB_R, 128), jnp.float32),
        mesh=mesh,
        scratch_types=[
            pltpu.VMEM((NRING, 16, UB), jnp.float32),
            pltpu.VMEM((NRING, UB * 17), jnp.float32),
            pltpu.VMEM((NRING, UB // 8, 128), jnp.float32),
            pltpu.VMEM((16, 128), jnp.float32),
            pltpu.SemaphoreType.DMA((NRING,)),
            pltpu.SemaphoreType.DMA((NRING,)),
            pltpu.SemaphoreType.DMA,
        ],
        compiler_params=pltpu.CompilerParams(needs_layout_passes=False),
    )
    def _sc_format(tabt_hbm, tailc_hbm, out_hbm, in_v, skw_v, tr_v, sm_v,
                   isems, osems, ssem):
        wid = lax.axis_index("s") * NC + lax.axis_index("c")
        iota = jnp.arange(16, dtype=jnp.int32)
        n_units = (BIGU - wid + NW - 1) // NW

        def unit_uid(k):
            return wid + k * NW

        def in_src(uid):
            f = uid // NBU
            t = lax.rem(uid, NBU)
            return tabt_hbm.at[pl.ds(pl.multiple_of(f * 16, 16), 16),
                               pl.ds(pl.multiple_of(t * UB, UB), UB)]

        def out_dst(uid):
            f = uid // NBU
            t = lax.rem(uid, NBU)
            return out_hbm.at[
                pl.ds(pl.multiple_of(f * VPF + t * (UB // 8), 8), UB // 8), :]

        def fire_in(k):
            slot = lax.rem(k, NRING)
            return pltpu.async_copy(in_src(unit_uid(k)), in_v.at[slot],
                                    isems.at[slot])

        for p in range(NRING - 1):
            fire_in(p)

        def unit_body(k, carry):
            slot = lax.rem(k, NRING)
            uid = unit_uid(k)

            @pl.when(k + NRING - 1 < n_units)
            def _():
                fire_in(k + NRING - 1)

            pltpu.make_async_copy(in_src(uid), in_v.at[slot],
                                  isems.at[slot]).wait()

            @pl.when(k >= NRING)
            def _():
                pltpu.make_async_copy(tr_v.at[slot],
                                      out_dst(unit_uid(k - NRING)),
                                      osems.at[slot]).wait()

            slotv = jnp.full((16,), slot, jnp.int32)
            iota17 = iota * 17

            def v0_body(v0, carry2):
                base = iota17 + v0 * (16 * 17)
                vals = [in_v[slot, e, pl.ds(v0 * 16, 16)] for e in range(16)]
                for e in range(16):
                    plsc.store_scatter(skw_v, [slotv, base + e], vals[e])
                return carry2

            lax.fori_loop(0, UB // 16, v0_body, 0)

            def vb_body(vb, carry2):
                vals = [skw_v[slot, pl.ds((vb * 8 + i) * 17, 16)]
                        for i in range(8)]
                for i in range(8):
                    tr_v[slot, vb, pl.ds(i * 16, 16)] = vals[i]
                return carry2

            lax.fori_loop(0, UB // 8, vb_body, 0)
            pltpu.async_copy(tr_v.at[slot], out_dst(uid), osems.at[slot])
            return carry

        lax.fori_loop(0, n_units, unit_body, 0)

        def drain_body(j, carry):
            pltpu.make_async_copy(tr_v.at[lax.rem(j, NRING)],
                                  out_dst(unit_uid(j)),
                                  osems.at[lax.rem(j, NRING)]).wait()
            return carry

        lax.fori_loop(lax.max(n_units - NRING, 0), n_units, drain_body, 0)

        @pl.when(wid < NUM_CAT)
        def _():
            f = wid
            pltpu.sync_copy(
                tabt_hbm.at[pl.ds(pl.multiple_of(f * 16, 16), 16),
                            pl.ds(SMALL0, 128)],
                sm_v)
            def vb_body(vb, carry2):
                vbase = jnp.full((16,), vb * 8, jnp.int32)
                vals = [plsc.load_gather(sm_v, [iota, vbase + dv])
                        for dv in range(8)]
                for dv in range(8):
                    tr_v[0, vb, pl.ds(dv * 16, 16)] = vals[dv]
                return carry2

            lax.fori_loop(0, 16, vb_body, 0)
            pltpu.sync_copy(
                tr_v.at[0, pl.ds(0, 16), :],
                out_hbm.at[
                    pl.ds(pl.multiple_of(f * VPF + SMALL0 // 8, 8), 16), :])
            pltpu.sync_copy(tailc_hbm.at[f], sm_v.at[pl.ds(0, 8), :])
            pltpu.sync_copy(
                sm_v.at[pl.ds(0, 8), :],
                out_hbm.at[
                    pl.ds(pl.multiple_of(f * VPF + TAIL0 // 8, 8), 8), :])

    return _sc_format


@functools.cache
def _make_sc_gather():
    mesh = plsc.VectorSubcoreMesh(core_axis_name="c", subcore_axis_name="s")

    @functools.partial(
        pl.kernel,
        out_type=jax.ShapeDtypeStruct((B, XD), jnp.float32),
        mesh=mesh,
        scratch_types=[
            pltpu.VMEM((GPW, 128), jnp.int32),
            pltpu.VMEM((GPW, 128), jnp.int32),
            pltpu.VMEM((4, 128, 128), jnp.float32),
            pltpu.VMEM((ROWS_F, XD), jnp.float32),
            pltpu.SemaphoreType.DMA((4,)),
        ],
        compiler_params=pltpu.CompilerParams(needs_layout_passes=False),
    )
    def _sc_gather(tab_hbm, idx_hbm, out_hbm, idx_v, idxr_v, rows_v, asm_v,
                   sems):
        wid = lax.axis_index("s") * NC + lax.axis_index("c")
        pltpu.sync_copy(idx_hbm.at[wid], idx_v)

        iota = jnp.arange(16, dtype=jnp.int32)
        zeros = jnp.zeros((16,), jnp.float32)

        def zero_body(r, carry):
            for c in range(EMB * NUM_CAT // 16, XD // 16):
                asm_v[r, pl.ds(c * 16, 16)] = zeros
            return carry

        lax.fori_loop(0, ROWS_F, zero_body, 0)

        def shift_body(g, carry):
            for c in range(8):
                idxr_v[g, pl.ds(c * 16, 16)] = (
                    idx_v[g, pl.ds(c * 16, 16)] >> 3)
            return carry

        lax.fori_loop(0, GPW, shift_body, 0)

        def fire(g, slot):
            return pltpu.async_copy(
                tab_hbm.at[idxr_v.at[g]], rows_v.at[slot], sems.at[slot])

        for p in range(3):
            fire(p, p)

        def group_body(g, carry):
            slot = lax.rem(g, 4)

            @pl.when(g + 3 < GPW)
            def _():
                fire(g + 3, lax.rem(g + 3, 4))

            pltpu.make_async_copy(
                tab_hbm.at[idxr_v.at[g]], rows_v.at[slot],
                sems.at[slot]).wait()

            base = lax.rem(g, FLUSH) * 128
            arow_base = jnp.full((16,), 0, jnp.int32)
            slotv = jnp.full((16,), slot, jnp.int32)
            for c in range(8):
                flat = base + c * 16 + iota
                idxc = idx_v[g, pl.ds(c * 16, 16)]
                off = (idxc & 7) << 4
                srow = flat // NUM_CAT
                scol = lax.rem(flat, NUM_CAT) << 4
                jvec = jnp.full((16,), c * 16, jnp.int32) + iota
                vals = [plsc.load_gather(rows_v, [slotv, jvec, off + e])
                        for e in range(EMB)]
                for e in range(EMB):
                    plsc.store_scatter(asm_v, [srow, scol + e], vals[e])

            @pl.when(lax.rem(g, FLUSH) == FLUSH - 1)
            def _():
                s = g // FLUSH
                pltpu.sync_copy(
                    asm_v,
                    out_hbm.at[pl.ds(wid * SPW + s * ROWS_F, ROWS_F)])

            return carry

        lax.fori_loop(0, GPW, group_body, 0)

    return _sc_gather


BS = 1024
NB = B // BS
_INV_B = 1.0 / B
_F32 = jnp.float32


def _a_body(nx_ref, em_ref, w1a_ref, w1b_ref, b1_ref, wwa_ref, wwb_ref,
            h1_ref, wide_ref, s1_ref, s2_ref):
    i = pl.program_id(0)
    nx = nx_ref[...]
    em = em_ref[...]
    h = (jnp.dot(nx, w1a_ref[...], preferred_element_type=_F32)
         + jnp.dot(em, w1b_ref[...], preferred_element_type=_F32)
         + b1_ref[...])
    h1_ref[...] = h
    wide_ref[...] = (jnp.sum(nx * wwa_ref[...], axis=1)
                     + jnp.sum(em * wwb_ref[...], axis=1))
    s1 = jnp.sum(h, axis=0, keepdims=True)
    s2 = jnp.sum(h * h, axis=0, keepdims=True)

    @pl.when(i == 0)
    def _():
        s1_ref[...] = s1
        s2_ref[...] = s2

    @pl.when(i > 0)
    def _():
        s1_ref[...] += s1
        s2_ref[...] += s2


_a_call = pl.pallas_call(
    _a_body,
    grid=(NB,),
    in_specs=[
        pl.BlockSpec((BS, NUM_NUMERIC), lambda i: (i, 0)),
        pl.BlockSpec((BS, XD), lambda i: (i, 0)),
        pl.BlockSpec((NUM_NUMERIC, H1), lambda i: (0, 0)),
        pl.BlockSpec((XD, H1), lambda i: (0, 0)),
        pl.BlockSpec((H1,), lambda i: (0,)),
        pl.BlockSpec((1, NUM_NUMERIC), lambda i: (0, 0)),
        pl.BlockSpec((1, XD), lambda i: (0, 0)),
    ],
    out_specs=[
        pl.BlockSpec((BS, H1), lambda i: (i, 0)),
        pl.BlockSpec((BS,), lambda i: (i,)),
        pl.BlockSpec((1, H1), lambda i: (0, 0)),
        pl.BlockSpec((1, H1), lambda i: (0, 0)),
    ],
    out_shape=[
        jax.ShapeDtypeStruct((B, H1), _F32),
        jax.ShapeDtypeStruct((B,), _F32),
        jax.ShapeDtypeStruct((1, H1), _F32),
        jax.ShapeDtypeStruct((1, H1), _F32),
    ],
)


def _b_body(h1_ref, s1_ref, s2_ref, g1_ref, be1_ref, w2_ref, b2_ref,
            h2_ref, t1_ref, t2_ref):
    i = pl.program_id(0)
    mu = s1_ref[...] * _INV_B
    var = s2_ref[...] * _INV_B - mu * mu
    hn = jnp.maximum(
        g1_ref[...] * (h1_ref[...] - mu) * lax.rsqrt(var + EPS) + be1_ref[...],
        0.0)
    h2 = jnp.dot(hn, w2_ref[...], preferred_element_type=_F32) + b2_ref[...]
    h2_ref[...] = h2
    t1 = jnp.sum(h2, axis=0, keepdims=True)
    t2 = jnp.sum(h2 * h2, axis=0, keepdims=True)

    @pl.when(i == 0)
    def _():
        t1_ref[...] = t1
        t2_ref[...] = t2

    @pl.when(i > 0)
    def _():
        t1_ref[...] += t1
        t2_ref[...] += t2


_b_call = pl.pallas_call(
    _b_body,
    grid=(NB,),
    in_specs=[
        pl.BlockSpec((BS, H1), lambda i: (i, 0)),
        pl.BlockSpec((1, H1), lambda i: (0, 0)),
        pl.BlockSpec((1, H1), lambda i: (0, 0)),
        pl.BlockSpec((H1,), lambda i: (0,)),
        pl.BlockSpec((H1,), lambda i: (0,)),
        pl.BlockSpec((H1, H2), lambda i: (0, 0)),
        pl.BlockSpec((H2,), lambda i: (0,)),
    ],
    out_specs=[
        pl.BlockSpec((BS, H2), lambda i: (i, 0)),
        pl.BlockSpec((1, H2), lambda i: (0, 0)),
        pl.BlockSpec((1, H2), lambda i: (0, 0)),
    ],
    out_shape=[
        jax.ShapeDtypeStruct((B, H2), _F32),
        jax.ShapeDtypeStruct((1, H2), _F32),
        jax.ShapeDtypeStruct((1, H2), _F32),
    ],
)


def _c_body(h2_ref, t1_ref, t2_ref, g2_ref, be2_ref, w3_ref, wide_ref,
            b3w_ref, out_ref):
    mu = t1_ref[...] * _INV_B
    var = t2_ref[...] * _INV_B - mu * mu
    hn = jnp.maximum(
        g2_ref[...] * (h2_ref[...] - mu) * lax.rsqrt(var + EPS) + be2_ref[...],
        0.0)
    out_ref[...] = (jnp.sum(hn * w3_ref[...], axis=1) + wide_ref[...]
                    + b3w_ref[0, 0])


_c_call = pl.pallas_call(
    _c_body,
    grid=(NB,),
    in_specs=[
        pl.BlockSpec((BS, H2), lambda i: (i, 0)),
        pl.BlockSpec((1, H2), lambda i: (0, 0)),
        pl.BlockSpec((1, H2), lambda i: (0, 0)),
        pl.BlockSpec((H2,), lambda i: (0,)),
        pl.BlockSpec((H2,), lambda i: (0,)),
        pl.BlockSpec((1, H2), lambda i: (0, 0)),
        pl.BlockSpec((BS,), lambda i: (i,)),
        pl.BlockSpec(memory_space=pltpu.SMEM),
    ],
    out_specs=pl.BlockSpec((BS,), lambda i: (i,)),
    out_shape=jax.ShapeDtypeStruct((B,), _F32),
)


def kernel(num_x, cat_x, tables, W1, b1, g1, be1, W2, b2, g2, be2, W3, b3,
           Ww, bw):
    tabt = tables.transpose(0, 2, 1).reshape(NUM_CAT * EMB, VOCAB)
    tailc = jnp.pad(tables[:, TAIL0:, :].reshape(NUM_CAT, 4, 128),
                    ((0, 0), (0, 4), (0, 0)))
    tab = _make_sc_format()(tabt, tailc)
    idx = (cat_x.astype(jnp.int32)
           + (jnp.arange(NUM_CAT, dtype=jnp.int32) * (VPF * 8))[None, :]
           ).reshape(NW, GPW, 128)
    em = _make_sc_gather()(tab, idx)
    w1a, w1b = W1[:NUM_NUMERIC], W1[NUM_NUMERIC:]
    w1bp = jnp.pad(w1b, ((0, XD - NUM_CAT * EMB), (0, 0)))
    wwa = Ww[:NUM_NUMERIC, 0][None, :]
    wwb = jnp.pad(Ww[NUM_NUMERIC:, 0][None, :],
                  ((0, 0), (0, XD - NUM_CAT * EMB)))
    w3row = W3[:, 0][None, :]
    b3w = (b3 + bw).reshape(1, 1)
    h1, wide, s1, s2 = _a_call(num_x, em, w1a, w1bp, b1, wwa, wwb)
    h2, t1, t2 = _b_call(h1, s1, s2, g1, be1, W2, b2)
    return _c_call(h2, t1, t2, g2, be2, w3row, wide, b3w)
